# baseline (speedup 1.0000x reference)
import functools

import jax
import jax.numpy as jnp
from jax import lax
from jax.experimental import pallas as pl
from jax.experimental.pallas import tpu as pltpu
from jax.experimental.pallas import tpu_sc as plsc

_NC = 2
_NS = 16
_L = 16
_NW = _NC * _NS
_UNROLL = 4
_SLOTS = 4


@functools.lru_cache(maxsize=None)
def _make_sc_gather(B, N, E):
    F = N * N
    assert B % (_SLOTS * _NW) == 0 and E % (_L * _UNROLL) == 0
    bpw = B // _NW
    chunks = E // _L
    mesh = plsc.VectorSubcoreMesh(core_axis_name="c", subcore_axis_name="s")

    @functools.partial(
        pl.kernel,
        out_type=jax.ShapeDtypeStruct((B, E), jnp.float32),
        mesh=mesh,
        compiler_params=pltpu.CompilerParams(needs_layout_passes=False),
        scratch_types=[
            pltpu.VMEM((E,), jnp.int32),
            pltpu.VMEM((E,), jnp.int32),
            pltpu.VMEM((E,), jnp.int32),
            [pltpu.VMEM((E,), jnp.float32)] * _SLOTS,
            [pltpu.SemaphoreType.DMA] * _SLOTS,
            [pltpu.SemaphoreType.DMA] * _SLOTS,
        ],
    )
    def k(x_hbm, rows_hbm, cols_hbm, out_hbm,
          idx_v, rows_v, cols_v, ovs, sgs, sss):
        wid = lax.axis_index("s") * _NC + lax.axis_index("c")
        base = wid * bpw

        pltpu.sync_copy(rows_hbm, rows_v)
        pltpu.sync_copy(cols_hbm, cols_v)

        @plsc.parallel_loop(0, chunks, 1, unroll=_UNROLL)
        def _(m):
            sl = pl.ds(m * _L, _L)
            idx_v[sl] = rows_v[sl] * N + cols_v[sl]

        def gather_start(b, s):
            src = x_hbm.at[pl.ds(b * F, F)].at[idx_v]
            pltpu.async_copy(src, ovs[s], sgs[s])

        for s in range(_SLOTS):
            gather_start(base + s, s)

        def batch_body(k_, carry):
            b0 = base + _SLOTS * k_
            for s in range(_SLOTS):
                pltpu.make_async_copy(
                    x_hbm.at[pl.ds(0, F)].at[idx_v], ovs[s], sgs[s]).wait()
                pltpu.async_copy(ovs[s], out_hbm.at[b0 + s], sss[s])
            for s in range(_SLOTS):
                pltpu.make_async_copy(ovs[s], out_hbm.at[b0 + s], sss[s]).wait()

                @pl.when(b0 + s + _SLOTS < base + bpw)
                def _():
                    gather_start(b0 + s + _SLOTS, s)

            return carry

        lax.fori_loop(0, bpw // _SLOTS, batch_body, 0)

    return k


def kernel(x, rows, cols):
    B, N, _ = x.shape
    xflat = x.reshape(B * N * N)
    k = _make_sc_gather(B, N, rows.shape[0])
    return k(xflat, rows.astype(jnp.int32), cols.astype(jnp.int32))


# speedup vs baseline: 2.1077x; 2.1077x over previous
import functools

import jax
import jax.numpy as jnp
from jax import lax
from jax.experimental import pallas as pl
from jax.experimental.pallas import tpu as pltpu
from jax.experimental.pallas import tpu_sc as plsc

_NC = 2
_NS = 16
_L = 16
_NW = _NC * _NS
_UNROLL = 4


@functools.lru_cache(maxsize=None)
def _make_sc_gather(B, N, E):
    F = N * N
    assert B % (2 * _NW) == 0 and E % (_L * _UNROLL) == 0
    bpw = B // _NW
    chunks = E // _L
    mesh = plsc.VectorSubcoreMesh(core_axis_name="c", subcore_axis_name="s")

    @functools.partial(
        pl.kernel,
        out_type=jax.ShapeDtypeStruct((B, E), jnp.float32),
        mesh=mesh,
        compiler_params=pltpu.CompilerParams(use_tc_tiling_on_sc=False),
        scratch_types=[
            pltpu.VMEM((E,), jnp.int32),
            pltpu.VMEM((E,), jnp.int32),
            pltpu.VMEM((E,), jnp.int32),
            pltpu.VMEM_SHARED((_NS, 2, F), jnp.float32),
            pltpu.VMEM((E,), jnp.float32),
            pltpu.VMEM((E,), jnp.float32),
            pltpu.SemaphoreType.DMA,
            pltpu.SemaphoreType.DMA,
            pltpu.SemaphoreType.DMA,
            pltpu.SemaphoreType.DMA,
            pltpu.SemaphoreType.DMA,
            pltpu.SemaphoreType.DMA,
        ],
    )
    def k(x_hbm, rows_hbm, cols_hbm, out_hbm,
          idx_v, rows_v, cols_v, xsh, ova, ovb,
          sia, sib, sga, sgb, soa, sob):
        sid = lax.axis_index("s")
        wid = sid * _NC + lax.axis_index("c")
        base = wid * bpw
        xva = xsh.at[sid, 0]
        xvb = xsh.at[sid, 1]

        pltpu.sync_copy(rows_hbm, rows_v)
        pltpu.sync_copy(cols_hbm, cols_v)

        @plsc.parallel_loop(0, chunks, 1, unroll=_UNROLL)
        def _(m):
            sl = pl.ds(m * _L, _L)
            idx_v[sl] = rows_v[sl] * N + cols_v[sl]

        def step(k_, b, xv, ov, si, sg, so):
            pltpu.make_async_copy(x_hbm.at[pl.ds(b * F, F)], xv, si).wait()

            @pl.when(k_ > 0)
            def _():
                pltpu.make_async_copy(ov, out_hbm.at[b], so).wait()

            pltpu.async_copy(xv.at[idx_v], ov, sg).wait()
            pltpu.async_copy(ov, out_hbm.at[b], so)

            @pl.when(b + 2 < base + bpw)
            def _():
                pltpu.async_copy(x_hbm.at[pl.ds((b + 2) * F, F)], xv, si)

        pltpu.async_copy(x_hbm.at[pl.ds(base * F, F)], xva, sia)
        pltpu.async_copy(x_hbm.at[pl.ds((base + 1) * F, F)], xvb, sib)

        def batch_body(k_, carry):
            step(k_, base + 2 * k_, xva, ova, sia, sga, soa)
            step(k_, base + 2 * k_ + 1, xvb, ovb, sib, sgb, sob)
            return carry

        lax.fori_loop(0, bpw // 2, batch_body, 0)
        pltpu.make_async_copy(ova, out_hbm.at[base], soa).wait()
        pltpu.make_async_copy(ovb, out_hbm.at[base], sob).wait()

    return k


def kernel(x, rows, cols):
    B, N, _ = x.shape
    k = _make_sc_gather(B, N, rows.shape[0])
    return k(x.reshape(-1), rows.astype(jnp.int32), cols.astype(jnp.int32))


# speedup vs baseline: 4.6441x; 2.2034x over previous
import functools

import jax
import jax.numpy as jnp
from jax import lax
from jax.experimental import pallas as pl
from jax.experimental.pallas import tpu as pltpu
from jax.experimental.pallas import tpu_sc as plsc

_NC = 2
_NS = 16
_L = 16
_NW = _NC * _NS
_UNROLL = 4


@functools.lru_cache(maxsize=None)
def _make_sc_gather(B, N, E):
    F = N * N
    assert B % (4 * _NW) == 0 and E % (_L * _UNROLL) == 0
    bpw = B // _NW
    chunks = E // _L
    mesh = plsc.VectorSubcoreMesh(core_axis_name="c", subcore_axis_name="s")

    @functools.partial(
        pl.kernel,
        out_type=jax.ShapeDtypeStruct((B * E,), jnp.float32),
        mesh=mesh,
        compiler_params=pltpu.CompilerParams(needs_layout_passes=False),
        scratch_types=[
            pltpu.VMEM((E,), jnp.int32),
            pltpu.VMEM((E,), jnp.int32),
            pltpu.VMEM((E,), jnp.int32),
            pltpu.VMEM((2 * F,), jnp.float32),
            pltpu.VMEM((2 * F,), jnp.float32),
            pltpu.VMEM((2 * E,), jnp.float32),
            pltpu.VMEM((2 * E,), jnp.float32),
            pltpu.SemaphoreType.DMA,
            pltpu.SemaphoreType.DMA,
            pltpu.SemaphoreType.DMA,
            pltpu.SemaphoreType.DMA,
        ],
    )
    def k(x_hbm, rows_hbm, cols_hbm, out_hbm,
          idx_v, rows_v, cols_v, xva, xvb, ova, ovb,
          sia, sib, soa, sob):
        wid = lax.axis_index("s") * _NC + lax.axis_index("c")
        base = wid * bpw

        pltpu.sync_copy(rows_hbm, rows_v)
        pltpu.sync_copy(cols_hbm, cols_v)

        @plsc.parallel_loop(0, chunks, 1, unroll=_UNROLL)
        def _(m):
            sl = pl.ds(m * _L, _L)
            idx_v[sl] = rows_v[sl] * N + cols_v[sl]

        def gather(xv, ov):
            @plsc.parallel_loop(0, chunks, 1, unroll=_UNROLL)
            def _(m):
                sl = pl.ds(m * _L, _L)
                iv = idx_v[sl]
                ov[sl] = plsc.load_gather(xv, [iv])
                ov[pl.ds(E + m * _L, _L)] = plsc.load_gather(xv, [iv + F])

        def step(k_, b, xv, ov, si, so):
            pltpu.make_async_copy(
                x_hbm.at[pl.ds(b * F, 2 * F)], xv, si).wait()

            @pl.when(k_ > 0)
            def _():
                pltpu.make_async_copy(
                    ov, out_hbm.at[pl.ds(b * E, 2 * E)], so).wait()

            gather(xv, ov)
            pltpu.async_copy(ov, out_hbm.at[pl.ds(b * E, 2 * E)], so)

            @pl.when(b + 4 < base + bpw)
            def _():
                pltpu.async_copy(
                    x_hbm.at[pl.ds((b + 4) * F, 2 * F)], xv, si)

        pltpu.async_copy(x_hbm.at[pl.ds(base * F, 2 * F)], xva, sia)
        pltpu.async_copy(x_hbm.at[pl.ds((base + 2) * F, 2 * F)], xvb, sib)

        def batch_body(k_, carry):
            step(k_, base + 4 * k_, xva, ova, sia, soa)
            step(k_, base + 4 * k_ + 2, xvb, ovb, sib, sob)
            return carry

        lax.fori_loop(0, bpw // 4, batch_body, 0)
        pltpu.make_async_copy(ova, out_hbm.at[pl.ds(base * E, 2 * E)], soa).wait()
        pltpu.make_async_copy(ovb, out_hbm.at[pl.ds(base * E, 2 * E)], sob).wait()

    return k


def kernel(x, rows, cols):
    B, N, _ = x.shape
    E = rows.shape[0]
    k = _make_sc_gather(B, N, E)
    flat = k(x.reshape(-1), rows.astype(jnp.int32), cols.astype(jnp.int32))
    return flat.reshape(B, E)


# speedup vs baseline: 6.1561x; 1.3256x over previous
import functools

import jax
import jax.numpy as jnp
from jax import lax
from jax.experimental import pallas as pl
from jax.experimental.pallas import tpu as pltpu
from jax.experimental.pallas import tpu_sc as plsc

_NC = 2
_NS = 16
_L = 16
_NW = _NC * _NS
_UNROLL = 4


@functools.lru_cache(maxsize=None)
def _make_sc_gather(B, N, E):
    F = N * N
    assert B % (4 * _NW) == 0 and E % (_L * _UNROLL) == 0
    bpw = B // _NW
    chunks = E // _L
    mesh = plsc.VectorSubcoreMesh(core_axis_name="c", subcore_axis_name="s")

    @functools.partial(
        pl.kernel,
        out_type=jax.ShapeDtypeStruct((B, E), jnp.float32),
        mesh=mesh,
        compiler_params=pltpu.CompilerParams(needs_layout_passes=False),
        scratch_types=[
            pltpu.VMEM((E,), jnp.int32),
            pltpu.VMEM((E,), jnp.int32),
            pltpu.VMEM((E,), jnp.int32),
            pltpu.VMEM((2 * F,), jnp.float32),
            pltpu.VMEM((2 * F,), jnp.float32),
            pltpu.VMEM((E,), jnp.float32),
            pltpu.VMEM((E,), jnp.float32),
            pltpu.VMEM((E,), jnp.float32),
            pltpu.VMEM((E,), jnp.float32),
            pltpu.SemaphoreType.DMA,
            pltpu.SemaphoreType.DMA,
            pltpu.SemaphoreType.DMA,
            pltpu.SemaphoreType.DMA,
        ],
    )
    def k(x_hbm, rows_hbm, cols_hbm, out_hbm,
          idx_v, rows_v, cols_v, xva, xvb, ova0, ova1, ovb0, ovb1,
          sia, sib, soa, sob):
        wid = lax.axis_index("s") * _NC + lax.axis_index("c")
        base = wid * bpw

        pltpu.sync_copy(rows_hbm, rows_v)
        pltpu.sync_copy(cols_hbm, cols_v)

        @plsc.parallel_loop(0, chunks, 1, unroll=_UNROLL)
        def _(m):
            sl = pl.ds(m * _L, _L)
            idx_v[sl] = rows_v[sl] * N + cols_v[sl]

        def gather(xv, ov0, ov1):
            @plsc.parallel_loop(0, chunks, 1, unroll=_UNROLL)
            def _(m):
                sl = pl.ds(m * _L, _L)
                iv = idx_v[sl]
                ov0[sl] = plsc.load_gather(xv, [iv])
                ov1[sl] = plsc.load_gather(xv, [iv + F])

        def step(k_, b, xv, ov0, ov1, si, so):
            pltpu.make_async_copy(
                x_hbm.at[pl.ds(b * F, 2 * F)], xv, si).wait()

            @pl.when(k_ > 0)
            def _():
                pltpu.make_async_copy(ov0, out_hbm.at[b], so).wait()
                pltpu.make_async_copy(ov1, out_hbm.at[b], so).wait()

            gather(xv, ov0, ov1)
            pltpu.async_copy(ov0, out_hbm.at[b], so)
            pltpu.async_copy(ov1, out_hbm.at[b + 1], so)

            @pl.when(b + 4 < base + bpw)
            def _():
                pltpu.async_copy(
                    x_hbm.at[pl.ds((b + 4) * F, 2 * F)], xv, si)

        pltpu.async_copy(x_hbm.at[pl.ds(base * F, 2 * F)], xva, sia)
        pltpu.async_copy(x_hbm.at[pl.ds((base + 2) * F, 2 * F)], xvb, sib)

        def batch_body(k_, carry):
            step(k_, base + 4 * k_, xva, ova0, ova1, sia, soa)
            step(k_, base + 4 * k_ + 2, xvb, ovb0, ovb1, sib, sob)
            return carry

        lax.fori_loop(0, bpw // 4, batch_body, 0)
        pltpu.make_async_copy(ova0, out_hbm.at[base], soa).wait()
        pltpu.make_async_copy(ova1, out_hbm.at[base], soa).wait()
        pltpu.make_async_copy(ovb0, out_hbm.at[base], sob).wait()
        pltpu.make_async_copy(ovb1, out_hbm.at[base], sob).wait()

    return k


def kernel(x, rows, cols):
    B, N, _ = x.shape
    E = rows.shape[0]
    k = _make_sc_gather(B, N, E)
    return k(x.reshape(-1), rows.astype(jnp.int32), cols.astype(jnp.int32))
